# Initial kernel scaffold; baseline (speedup 1.0000x reference)
import jax
import jax.numpy as jnp
from jax.experimental import pallas as pl


def kernel(protein_x, ligand_x, Wp1, bp1, Wp2, bp2, Wp3, bp3, Wl1, bl1, Wl2, bl2, Wl3, bl3, Wf1, bf1, Wf2, bf2, Wo, bo, protein_edge_index, protein_x_batch, ligand_edge_index, ligand_x_batch):
    raise NotImplementedError("write your pallas kernel here")


# speedup vs baseline: 1.4165x; 1.4165x over previous
import functools

import jax
import jax.numpy as jnp
from jax import lax
from jax.experimental import pallas as pl
from jax.experimental.pallas import tpu as pltpu

NREAL = 10000
NPAD = 10240
BLK = 512
GRID = NPAD // BLK

_INTERP = False


def _row_spec(f):
    return pl.BlockSpec((BLK, f), lambda i: (i, 0))


def _full_spec(shape):
    nd = len(shape)
    return pl.BlockSpec(shape, lambda *a: (0,) * nd)


def _l1_body(x, W, d0, d1, dis_o, hp_o, lo_o, hi_o):
    i = pl.program_id(0)
    rows = i * BLK + lax.broadcasted_iota(jnp.int32, (BLK, 1), 0)
    deg = d0[:, :1] + d1[:, :1] + 1.0
    dis = jnp.where(rows < NREAL, lax.rsqrt(deg), 0.0)
    h = jnp.dot(x[...], W[...], preferred_element_type=jnp.float32, precision=lax.Precision.HIGHEST)
    hp = h * dis
    f = hp.shape[1]
    dis_o[...] = jnp.broadcast_to(dis, (BLK, 8))
    hp_o[...] = hp
    lo_o[...] = hp[:, : f // 2]
    hi_o[...] = hp[:, f // 2:]


def _mid_body(alo, ahi, hp, dis, b, W, hpn_o, lo_o, hi_o):
    A = jnp.concatenate([alo[...], ahi[...]], axis=1)
    d = dis[:, :1]
    x = jnp.maximum((A + hp[...]) * d + b[...], 0.0)
    h = jnp.dot(x, W[...], preferred_element_type=jnp.float32, precision=lax.Precision.HIGHEST)
    hpn = h * d
    f = hpn.shape[1]
    hpn_o[...] = hpn
    lo_o[...] = hpn[:, : f // 2]
    hi_o[...] = hpn[:, f // 2:]


def _pool_body(alo, ahi, hp, dis, b, bat, ps_o, cnt_o):
    i = pl.program_id(0)
    A = jnp.concatenate([alo[...], ahi[...]], axis=1)
    x = jnp.maximum((A + hp[...]) * dis[:, :1] + b[...], 0.0)
    g = lax.broadcasted_iota(jnp.int32, (BLK, 64), 1)
    onehot = (bat[...] == g).astype(jnp.float32)
    ps = lax.dot_general(onehot, x, (((0,), (0,)), ((), ())),
                         preferred_element_type=jnp.float32, precision=lax.Precision.HIGHEST)
    c = jnp.sum(onehot, axis=0)

    @pl.when(i == 0)
    def _():
        ps_o[...] = jnp.zeros_like(ps_o)
        cnt_o[...] = jnp.zeros_like(cnt_o)

    ps_o[...] += ps
    cnt_o[...] += jnp.broadcast_to(c[:, None], cnt_o.shape)


def _mlp_body(pp, cp, plg, cl, Wf1, bf1, Wf2, bf2, Wo, bo, o):
    xp = pp[...] / jnp.maximum(cp[:, :1], 1.0)
    xl = plg[...] / jnp.maximum(cl[:, :1], 1.0)
    x = jnp.concatenate([xp, xl], axis=1)
    y = jnp.maximum(
        jnp.dot(x, Wf1[...], preferred_element_type=jnp.float32, precision=lax.Precision.HIGHEST) + bf1[...], 0.0)
    z = jnp.maximum(
        jnp.dot(y, Wf2[...], preferred_element_type=jnp.float32, precision=lax.Precision.HIGHEST) + bf2[...], 0.0)
    o[...] = jnp.dot(z, Wo[...], preferred_element_type=jnp.float32, precision=lax.Precision.HIGHEST) + bo[...]


def _l1_call(xp, W, deg0, deg1):
    fin, f = W.shape
    return pl.pallas_call(
        _l1_body,
        grid=(GRID,),
        in_specs=[_row_spec(fin), _full_spec(W.shape), _row_spec(8), _row_spec(8)],
        out_specs=[_row_spec(8), _row_spec(f), _row_spec(f // 2), _row_spec(f // 2)],
        out_shape=[
            jax.ShapeDtypeStruct((NPAD, 8), jnp.float32),
            jax.ShapeDtypeStruct((NPAD, f), jnp.float32),
            jax.ShapeDtypeStruct((NPAD, f // 2), jnp.float32),
            jax.ShapeDtypeStruct((NPAD, f // 2), jnp.float32),
        ],
        interpret=_INTERP,
    )(xp, W, deg0, deg1)


def _mid_call(alo, ahi, hp, dis, b, W):
    fin, f = W.shape
    return pl.pallas_call(
        _mid_body,
        grid=(GRID,),
        in_specs=[_row_spec(fin // 2), _row_spec(fin // 2), _row_spec(fin),
                  _row_spec(8), _full_spec((1, fin)), _full_spec(W.shape)],
        out_specs=[_row_spec(f), _row_spec(f // 2), _row_spec(f // 2)],
        out_shape=[
            jax.ShapeDtypeStruct((NPAD, f), jnp.float32),
            jax.ShapeDtypeStruct((NPAD, f // 2), jnp.float32),
            jax.ShapeDtypeStruct((NPAD, f // 2), jnp.float32),
        ],
        interpret=_INTERP,
    )(alo, ahi, hp, dis, b.reshape(1, fin), W)


def _pool_call(alo, ahi, hp, dis, b, batch):
    f = hp.shape[1]
    return pl.pallas_call(
        _pool_body,
        grid=(GRID,),
        in_specs=[_row_spec(f // 2), _row_spec(f // 2), _row_spec(f),
                  _row_spec(8), _full_spec((1, f)), _row_spec(1)],
        out_specs=[_full_spec((64, f)), _full_spec((64, 128))],
        out_shape=[
            jax.ShapeDtypeStruct((64, f), jnp.float32),
            jax.ShapeDtypeStruct((64, 128), jnp.float32),
        ],
        compiler_params=pltpu.CompilerParams(
            dimension_semantics=("arbitrary",)),
        interpret=_INTERP,
    )(alo, ahi, hp, dis, b.reshape(1, f), batch)


def _mlp_call(pp, cp, plg, cl, Wf1, bf1, Wf2, bf2, Wo, bo):
    return pl.pallas_call(
        _mlp_body,
        in_specs=[_full_spec(pp.shape), _full_spec(cp.shape),
                  _full_spec(plg.shape), _full_spec(cl.shape),
                  _full_spec(Wf1.shape), _full_spec((1, 1024)),
                  _full_spec(Wf2.shape), _full_spec((1, 512)),
                  _full_spec(Wo.shape), _full_spec((1, 1))],
        out_specs=_full_spec((64, 1)),
        out_shape=jax.ShapeDtypeStruct((64, 1), jnp.float32),
        interpret=_INTERP,
    )(pp, cp, plg, cl, Wf1, bf1.reshape(1, 1024), Wf2, bf2.reshape(1, 512),
      Wo, bo.reshape(1, 1))


def _sc_deg(dst):
    deg = jnp.zeros((NPAD,), jnp.float32).at[dst.reshape(-1)].add(1.0)
    degb = jnp.broadcast_to(deg[:, None], (NPAD, 8))
    half = jnp.broadcast_to(jnp.zeros((NPAD, 1), jnp.float32), (NPAD, 8))
    return degb, half


def _sc_scatter(hplo, hphi, src, dst):
    s = src.reshape(-1)
    d = dst.reshape(-1)
    alo = jnp.zeros_like(hplo).at[d].add(hplo[s])
    ahi = jnp.zeros_like(hphi).at[d].add(hphi[s])
    return alo, ahi


def _prep_edges(ei):
    E = ei.shape[1]
    Epad = ((E + 2047) // 2048) * 2048
    pad = jnp.full((2, Epad - E), NREAL, jnp.int32)
    eip = jnp.concatenate([ei.astype(jnp.int32), pad], axis=1)
    K = Epad // 2048
    src = eip[0].reshape(16, K, 128)
    dst = eip[1].reshape(16, K, 128)
    return src, dst


def _branch(x, ei, W1, b1, W2, b2, W3, b3, batch):
    xp = jnp.pad(x, ((0, NPAD - NREAL), (0, 0)))
    batchp = jnp.pad(batch.astype(jnp.int32), (0, NPAD - NREAL),
                     constant_values=64).reshape(NPAD, 1)
    src, dst = _prep_edges(ei)

    deg0, deg1 = _sc_deg(dst)
    dis, hp1, hp1lo, hp1hi = _l1_call(xp, W1, deg0, deg1)
    a1lo, a1hi = _sc_scatter(hp1lo, hp1hi, src, dst)
    hp2, hp2lo, hp2hi = _mid_call(a1lo, a1hi, hp1, dis, b1, W2)
    a2lo, a2hi = _sc_scatter(hp2lo, hp2hi, src, dst)
    hp3, hp3lo, hp3hi = _mid_call(a2lo, a2hi, hp2, dis, b2, W3)
    a3lo, a3hi = _sc_scatter(hp3lo, hp3hi, src, dst)
    return _pool_call(a3lo, a3hi, hp3, dis, b3, batchp)


def kernel(protein_x, ligand_x, Wp1, bp1, Wp2, bp2, Wp3, bp3, Wl1, bl1,
           Wl2, bl2, Wl3, bl3, Wf1, bf1, Wf2, bf2, Wo, bo,
           protein_edge_index, protein_x_batch, ligand_edge_index,
           ligand_x_batch):
    pp, cp = _branch(protein_x, protein_edge_index, Wp1, bp1, Wp2, bp2,
                     Wp3, bp3, protein_x_batch)
    plg, cl = _branch(ligand_x, ligand_edge_index, Wl1, bl1, Wl2, bl2,
                      Wl3, bl3, ligand_x_batch)
    return _mlp_call(pp, cp, plg, cl, Wf1, bf1, Wf2, bf2, Wo, bo)
